# baseline (speedup 1.0000x reference)
import functools

import jax
import jax.numpy as jnp
from jax import lax
from jax.experimental import pallas as pl
from jax.experimental.pallas import tpu as pltpu
from jax.experimental.pallas import tpu_sc as plsc

N = 10000
E = 100000
D_IN = 128
D_H = 256
N_CLS = 5

NC = 2
NS = 16
NW = NC * NS
N_PAD = 10240
TRASH = N
E_PAD = 102400
PER_SUB = E_PAD // NW
G = 320
NSTEP = PER_SUB // G
RPT = N_PAD // NS

BN = 1000
NB = N // BN


def _seg_sums(xflat, zeros, srcs_list, dsts):
    L = len(srcs_list)
    mesh = plsc.VectorSubcoreMesh(core_axis_name="c", subcore_axis_name="s")
    out_type = jax.ShapeDtypeStruct((L, 3, NC, N_PAD, 128), jnp.float32)
    scratch = [
        pltpu.VMEM((1, G), jnp.int32),
        pltpu.VMEM((1, G), jnp.int32),
        pltpu.VMEM((G, 128), jnp.float32),
        pltpu.VMEM_SHARED((N_PAD, 128), jnp.float32),
    ]

    @functools.partial(pl.kernel, out_type=out_type, mesh=mesh,
                       scratch_types=scratch)
    def k(x_hbm, z_hbm, *rest):
        srcs_hbms = rest[:L]
        dsts_hbm, sums_hbm, sidx, didx, rows, acc = rest[L:]
        cid = lax.axis_index("c")
        sid = lax.axis_index("s")
        rbase = sid * RPT

        for l in range(L):
            for t in range(3):
                pltpu.sync_copy(z_hbm.at[pl.ds(rbase, RPT)],
                                acc.at[pl.ds(rbase, RPT)])
                plsc.subcore_barrier()

                @pl.loop(0, NSTEP)
                def _(i):
                    pltpu.sync_copy(srcs_hbms[l].at[t, cid, sid, i], sidx)
                    pltpu.sync_copy(dsts_hbm.at[t, cid, sid, i], didx)
                    pltpu.sync_copy(x_hbm.at[sidx.at[0]], rows)
                    pltpu.sync_copy(rows, acc.at[didx.at[0]], add=True)
                plsc.subcore_barrier()

                pltpu.sync_copy(acc.at[pl.ds(rbase, RPT)],
                                sums_hbm.at[l, t, cid, pl.ds(rbase, RPT)])
                plsc.subcore_barrier()

    return k(xflat, zeros, *srcs_list, dsts)


def _seg_counts(zeros, ones, dsts):
    mesh = plsc.VectorSubcoreMesh(core_axis_name="c", subcore_axis_name="s")
    out_type = jax.ShapeDtypeStruct((3, NC, N_PAD, 128), jnp.float32)
    scratch = [
        pltpu.VMEM((G, 128), jnp.float32),
        pltpu.VMEM((1, G), jnp.int32),
        pltpu.VMEM_SHARED((N_PAD, 128), jnp.float32),
    ]

    @functools.partial(pl.kernel, out_type=out_type, mesh=mesh,
                       scratch_types=scratch)
    def k(z_hbm, ones_hbm, dsts_hbm, cnt_hbm, ones, didx, cacc):
        cid = lax.axis_index("c")
        sid = lax.axis_index("s")
        rbase = sid * RPT

        pltpu.sync_copy(ones_hbm, ones)

        for t in range(3):
            pltpu.sync_copy(z_hbm.at[pl.ds(rbase, RPT)],
                            cacc.at[pl.ds(rbase, RPT)])
            plsc.subcore_barrier()

            @pl.loop(0, NSTEP)
            def _(i):
                pltpu.sync_copy(dsts_hbm.at[t, cid, sid, i], didx)
                pltpu.sync_copy(ones, cacc.at[didx.at[0]], add=True)
            plsc.subcore_barrier()

            pltpu.sync_copy(cacc.at[pl.ds(rbase, RPT)],
                            cnt_hbm.at[t, cid, pl.ds(rbase, RPT)])
            plsc.subcore_barrier()

    return k(zeros, ones, dsts)


def _p0_body(x_ref, w_ref, b_ref, o_ref):
    t = pl.program_id(0)
    o_ref[0] = jax.nn.relu(
        jnp.dot(x_ref[...], w_ref[0], preferred_element_type=jnp.float32)
        + b_ref[t])


def _project0(x, Wp0, bp0):
    return pl.pallas_call(
        _p0_body,
        grid=(3, NB),
        in_specs=[
            pl.BlockSpec((BN, D_IN), lambda t, i: (i, 0)),
            pl.BlockSpec((1, D_IN, D_IN), lambda t, i: (t, 0, 0)),
            pl.BlockSpec((3, D_IN), lambda t, i: (0, 0)),
        ],
        out_specs=pl.BlockSpec((1, BN, D_IN), lambda t, i: (t, i, 0)),
        out_shape=jax.ShapeDtypeStruct((3, N, D_IN), jnp.float32),
    )(x, Wp0, bp0)


def _combine_body(n_chunk, normalize, relu_ln, *refs):
    sums_refs = refs[:n_chunk]
    (cnt_ref, x_ref, wl_ref, bl_ref, wr_ref, g_ref, b_ref, h_ref,
     hs_ref) = refs[n_chunk:]
    x = x_ref[...]
    acc = jnp.zeros((BN, D_H), jnp.float32)
    for t in range(3):
        cnt = cnt_ref[t, 0, :, 0] + cnt_ref[t, 1, :, 0]
        cnt = jnp.maximum(cnt, 1.0)
        parts = []
        for c in range(n_chunk):
            s = sums_refs[c][0, t, 0] + sums_refs[c][0, t, 1]
            parts.append(s / cnt[:, None])
        aggr = jnp.concatenate(parts, axis=1) if n_chunk > 1 else parts[0]
        o = (jnp.dot(aggr, wl_ref[t], preferred_element_type=jnp.float32)
             + bl_ref[t]
             + jnp.dot(x, wr_ref[t], preferred_element_type=jnp.float32))
        if normalize:
            nrm = jnp.sqrt(jnp.sum(o * o, axis=-1, keepdims=True))
            o = o / jnp.maximum(nrm, 1e-12)
        acc = acc + o
    h = acc / 3.0
    if relu_ln:
        h = jax.nn.relu(h)
        mu = jnp.mean(h, axis=-1, keepdims=True)
        var = jnp.mean((h - mu) ** 2, axis=-1, keepdims=True)
        h = (h - mu) / jnp.sqrt(var + 1e-5) * g_ref[0] + b_ref[0]
    h_ref[...] = h
    hs_ref[0] = h[:, :128]
    hs_ref[1] = h[:, 128:]


def _combine(sums, n_chunk, cnt, x, Wl, bl, Wr, ln_g, ln_b, normalize,
             relu_ln):
    d_in = x.shape[1]
    body = functools.partial(_combine_body, n_chunk, normalize, relu_ln)
    sum_specs = [
        pl.BlockSpec((1, 3, NC, BN, 128), lambda i, c=c: (c, 0, 0, i, 0))
        for c in range(n_chunk)]
    return pl.pallas_call(
        body,
        grid=(NB,),
        in_specs=sum_specs + [
            pl.BlockSpec((3, NC, BN, 128), lambda i: (0, 0, i, 0)),
            pl.BlockSpec((BN, d_in), lambda i: (i, 0)),
            pl.BlockSpec((3, d_in, D_H), lambda i: (0, 0, 0)),
            pl.BlockSpec((3, D_H), lambda i: (0, 0)),
            pl.BlockSpec((3, d_in, D_H), lambda i: (0, 0, 0)),
            pl.BlockSpec((1, D_H), lambda i: (0, 0)),
            pl.BlockSpec((1, D_H), lambda i: (0, 0)),
        ],
        out_specs=[
            pl.BlockSpec((BN, D_H), lambda i: (i, 0)),
            pl.BlockSpec((2, BN, 128), lambda i: (0, i, 0)),
        ],
        out_shape=[
            jax.ShapeDtypeStruct((N, D_H), jnp.float32),
            jax.ShapeDtypeStruct((2, N, 128), jnp.float32),
        ],
    )(*([sums] * n_chunk), cnt, x, Wl, bl, Wr, ln_g, ln_b)


def _final_body(s0_ref, s1_ref, cnt_ref, x_ref, wl_ref, bl_ref, wr_ref,
                cW1_ref, cb1_ref, cW2_ref, cb2_ref, g_ref, b_ref,
                cW3_ref, cb3_ref, o_ref):
    x = x_ref[...]
    sums_refs = (s0_ref, s1_ref)
    acc = jnp.zeros((BN, D_H), jnp.float32)
    for t in range(3):
        cnt = cnt_ref[t, 0, :, 0] + cnt_ref[t, 1, :, 0]
        cnt = jnp.maximum(cnt, 1.0)
        parts = []
        for c in range(2):
            s = sums_refs[c][0, t, 0] + sums_refs[c][0, t, 1]
            parts.append(s / cnt[:, None])
        aggr = jnp.concatenate(parts, axis=1)
        acc = acc + (
            jnp.dot(aggr, wl_ref[t], preferred_element_type=jnp.float32)
            + bl_ref[t]
            + jnp.dot(x, wr_ref[t], preferred_element_type=jnp.float32))
    h = acc / 3.0
    z = jax.nn.relu(
        jnp.dot(h, cW1_ref[...], preferred_element_type=jnp.float32)
        + cb1_ref[...])
    z = jax.nn.relu(
        jnp.dot(z, cW2_ref[...], preferred_element_type=jnp.float32)
        + cb2_ref[...])
    mu = jnp.mean(z, axis=-1, keepdims=True)
    var = jnp.mean((z - mu) ** 2, axis=-1, keepdims=True)
    z = (z - mu) / jnp.sqrt(var + 1e-5) * g_ref[...] + b_ref[...]
    o_ref[...] = (jnp.dot(z, cW3_ref[...], preferred_element_type=jnp.float32)
                  + cb3_ref[...])


def _final(sums, cnt, h, Wl, bl, Wr, cW1, cb1, cW2, cb2, cg, cb, cW3,
           cb3):
    dh2 = D_H // 2
    return pl.pallas_call(
        _final_body,
        grid=(NB,),
        in_specs=[
            pl.BlockSpec((1, 3, NC, BN, 128), lambda i: (0, 0, 0, i, 0)),
            pl.BlockSpec((1, 3, NC, BN, 128), lambda i: (1, 0, 0, i, 0)),
            pl.BlockSpec((3, NC, BN, 128), lambda i: (0, 0, i, 0)),
            pl.BlockSpec((BN, D_H), lambda i: (i, 0)),
            pl.BlockSpec((3, D_H, D_H), lambda i: (0, 0, 0)),
            pl.BlockSpec((3, D_H), lambda i: (0, 0)),
            pl.BlockSpec((3, D_H, D_H), lambda i: (0, 0, 0)),
            pl.BlockSpec((D_H, dh2), lambda i: (0, 0)),
            pl.BlockSpec((dh2,), lambda i: (0,)),
            pl.BlockSpec((dh2, dh2), lambda i: (0, 0)),
            pl.BlockSpec((dh2,), lambda i: (0,)),
            pl.BlockSpec((dh2,), lambda i: (0,)),
            pl.BlockSpec((dh2,), lambda i: (0,)),
            pl.BlockSpec((dh2, N_CLS), lambda i: (0, 0)),
            pl.BlockSpec((N_CLS,), lambda i: (0,)),
        ],
        out_specs=pl.BlockSpec((BN, N_CLS), lambda i: (i, 0)),
        out_shape=jax.ShapeDtypeStruct((N, N_CLS), jnp.float32),
    )(sums, sums, cnt, h, Wl, bl, Wr, cW1, cb1, cW2, cb2, cg, cb, cW3, cb3)


def kernel(x, ei, Wp0, bp0, Wl0, bl0, Wr0, Wl1, bl1, Wr1, Wl2, bl2, Wr2,
           ln_g, ln_b, cW1, cb1, cW2, cb2, cln_g, cln_b, cW3, cb3):
    src = ei[:, 0, :]
    dst = ei[:, 1, :]
    src_pad = jnp.pad(src, ((0, 0), (0, E_PAD - E)))
    dst_pad = jnp.pad(dst, ((0, 0), (0, E_PAD - E)), constant_values=TRASH)
    srcs0 = (src_pad + (jnp.arange(3, dtype=jnp.int32) * N)[:, None])
    srcs0 = srcs0.reshape(3, NC, NS, NSTEP, 1, G)
    srcsH0 = src_pad.reshape(3, NC, NS, NSTEP, 1, G)
    srcsH1 = (src_pad + N).reshape(3, NC, NS, NSTEP, 1, G)
    dst_blk = dst_pad.reshape(3, NC, NS, NSTEP, 1, G)

    zeros = jnp.zeros((N_PAD, 128), jnp.float32)
    ones = jnp.ones((G, 128), jnp.float32)
    cnt = _seg_counts(zeros, ones, dst_blk)

    xs = _project0(x, Wp0, bp0)
    sums0 = _seg_sums(xs.reshape(3 * N, D_IN), zeros, [srcs0], dst_blk)
    h, hs = _combine(sums0, 1, cnt, x, Wl0, bl0, Wr0, ln_g[0:1], ln_b[0:1],
                     normalize=True, relu_ln=True)

    hsf = hs.reshape(2 * N, 128)
    sums1 = _seg_sums(hsf, zeros, [srcsH0, srcsH1], dst_blk)
    h2, h2s = _combine(sums1, 2, cnt, h, Wl1, bl1, Wr1, ln_g[1:2],
                       ln_b[1:2], normalize=False, relu_ln=True)

    h2sf = h2s.reshape(2 * N, 128)
    sums2 = _seg_sums(h2sf, zeros, [srcsH0, srcsH1], dst_blk)
    return _final(sums2, cnt, h2, Wl2, bl2, Wr2, cW1, cb1, cW2,
                  cb2, cln_g, cln_b, cW3, cb3)


# speedup vs baseline: 1.3166x; 1.3166x over previous
import functools

import jax
import jax.numpy as jnp
from jax import lax
from jax.experimental import pallas as pl
from jax.experimental.pallas import tpu as pltpu
from jax.experimental.pallas import tpu_sc as plsc

N = 10000
E = 100000
D_IN = 128
D_H = 256
N_CLS = 5

NC = 2
NS = 16
NW = NC * NS
N_PAD = 10240
TRASH = N
E_PAD = 102400
PER_SUB = E_PAD // NW
G = 320
NSTEP = PER_SUB // G
NSTEP2 = E_PAD // NS // G
RPT = N_PAD // NS

BN = 1000
NB = N // BN


def _seg_sums(xflat, zeros, srcs_list, dsts):
    L = len(srcs_list)
    mesh = plsc.VectorSubcoreMesh(core_axis_name="c", subcore_axis_name="s")
    out_type = jax.ShapeDtypeStruct((L, 3, NC, N_PAD, 128), jnp.float32)
    scratch = [
        pltpu.VMEM((1, G), jnp.int32),
        pltpu.VMEM((1, G), jnp.int32),
        pltpu.VMEM((G, 128), jnp.float32),
        pltpu.VMEM_SHARED((N_PAD, 128), jnp.float32),
    ]

    @functools.partial(pl.kernel, out_type=out_type, mesh=mesh,
                       scratch_types=scratch)
    def k(x_hbm, z_hbm, *rest):
        srcs_hbms = rest[:L]
        dsts_hbm, sums_hbm, sidx, didx, rows, acc = rest[L:]
        cid = lax.axis_index("c")
        sid = lax.axis_index("s")
        rbase = sid * RPT

        for l in range(L):
            for t in range(3):
                pltpu.sync_copy(z_hbm.at[pl.ds(rbase, RPT)],
                                acc.at[pl.ds(rbase, RPT)])
                plsc.subcore_barrier()

                @pl.loop(0, NSTEP)
                def _(i):
                    pltpu.sync_copy(srcs_hbms[l].at[t, cid, sid, i], sidx)
                    pltpu.sync_copy(dsts_hbm.at[t, cid, sid, i], didx)
                    pltpu.sync_copy(x_hbm.at[sidx.at[0]], rows)
                    pltpu.sync_copy(rows, acc.at[didx.at[0]], add=True)
                plsc.subcore_barrier()

                pltpu.sync_copy(acc.at[pl.ds(rbase, RPT)],
                                sums_hbm.at[l, t, cid, pl.ds(rbase, RPT)])
                plsc.subcore_barrier()

    return k(xflat, zeros, *srcs_list, dsts)


def _seg_sums_ch(xflat, zeros, srcs, dsts):
    mesh = plsc.VectorSubcoreMesh(core_axis_name="c", subcore_axis_name="s")
    out_type = jax.ShapeDtypeStruct((3, NC, N_PAD, 128), jnp.float32)
    scratch = [
        pltpu.VMEM((1, G), jnp.int32),
        pltpu.VMEM((1, G), jnp.int32),
        pltpu.VMEM((G, 128), jnp.float32),
        pltpu.VMEM_SHARED((N_PAD, 128), jnp.float32),
    ]

    @functools.partial(pl.kernel, out_type=out_type, mesh=mesh,
                       scratch_types=scratch)
    def k(x_hbm, z_hbm, srcs_hbm, dsts_hbm, sums_hbm, sidx, didx, rows, acc):
        cid = lax.axis_index("c")
        sid = lax.axis_index("s")
        rbase = sid * RPT

        for t in range(3):
            pltpu.sync_copy(z_hbm.at[pl.ds(rbase, RPT)],
                            acc.at[pl.ds(rbase, RPT)])
            plsc.subcore_barrier()

            @pl.loop(0, NSTEP2)
            def _(i):
                pltpu.sync_copy(srcs_hbm.at[cid, t, sid, i], sidx)
                pltpu.sync_copy(dsts_hbm.at[t, sid, i], didx)
                pltpu.sync_copy(x_hbm.at[sidx.at[0]], rows)
                pltpu.sync_copy(rows, acc.at[didx.at[0]], add=True)
            plsc.subcore_barrier()

            pltpu.sync_copy(acc.at[pl.ds(rbase, RPT)],
                            sums_hbm.at[t, cid, pl.ds(rbase, RPT)])
            plsc.subcore_barrier()

    return k(xflat, zeros, srcs, dsts)


def _seg_counts(zeros, ones, dsts):
    mesh = plsc.VectorSubcoreMesh(core_axis_name="c", subcore_axis_name="s")
    out_type = jax.ShapeDtypeStruct((3, NC, N_PAD, 128), jnp.float32)
    scratch = [
        pltpu.VMEM((G, 128), jnp.float32),
        pltpu.VMEM((1, G), jnp.int32),
        pltpu.VMEM_SHARED((N_PAD, 128), jnp.float32),
    ]

    @functools.partial(pl.kernel, out_type=out_type, mesh=mesh,
                       scratch_types=scratch)
    def k(z_hbm, ones_hbm, dsts_hbm, cnt_hbm, ones, didx, cacc):
        cid = lax.axis_index("c")
        sid = lax.axis_index("s")
        rbase = sid * RPT

        pltpu.sync_copy(ones_hbm, ones)

        for t in range(3):
            pltpu.sync_copy(z_hbm.at[pl.ds(rbase, RPT)],
                            cacc.at[pl.ds(rbase, RPT)])
            plsc.subcore_barrier()

            @pl.loop(0, NSTEP)
            def _(i):
                pltpu.sync_copy(dsts_hbm.at[t, cid, sid, i], didx)
                pltpu.sync_copy(ones, cacc.at[didx.at[0]], add=True)
            plsc.subcore_barrier()

            pltpu.sync_copy(cacc.at[pl.ds(rbase, RPT)],
                            cnt_hbm.at[t, cid, pl.ds(rbase, RPT)])
            plsc.subcore_barrier()

    return k(zeros, ones, dsts)


def _p0_body(x_ref, w_ref, b_ref, o_ref):
    t = pl.program_id(0)
    o_ref[0] = jax.nn.relu(
        jnp.dot(x_ref[...], w_ref[0], preferred_element_type=jnp.float32)
        + b_ref[t])


def _project0(x, Wp0, bp0):
    return pl.pallas_call(
        _p0_body,
        grid=(3, NB),
        in_specs=[
            pl.BlockSpec((BN, D_IN), lambda t, i: (i, 0)),
            pl.BlockSpec((1, D_IN, D_IN), lambda t, i: (t, 0, 0)),
            pl.BlockSpec((3, D_IN), lambda t, i: (0, 0)),
        ],
        out_specs=pl.BlockSpec((1, BN, D_IN), lambda t, i: (t, i, 0)),
        out_shape=jax.ShapeDtypeStruct((3, N, D_IN), jnp.float32),
    )(x, Wp0, bp0)


def _combine_body(n_chunk, chunked, normalize, relu_ln, *refs):
    sums_refs = refs[:n_chunk]
    (cnt_ref, x_ref, wl_ref, bl_ref, wr_ref, g_ref, b_ref, h_ref,
     hs_ref) = refs[n_chunk:]
    x = x_ref[...]
    acc = jnp.zeros((BN, D_H), jnp.float32)
    for t in range(3):
        cnt = cnt_ref[t, 0, :, 0] + cnt_ref[t, 1, :, 0]
        cnt = jnp.maximum(cnt, 1.0)
        parts = []
        if chunked:
            for c in range(2):
                parts.append(sums_refs[0][t, c] / cnt[:, None])
        else:
            for c in range(n_chunk):
                s = sums_refs[c][0, t, 0] + sums_refs[c][0, t, 1]
                parts.append(s / cnt[:, None])
        aggr = jnp.concatenate(parts, axis=1) if len(parts) > 1 else parts[0]
        o = (jnp.dot(aggr, wl_ref[t], preferred_element_type=jnp.float32)
             + bl_ref[t]
             + jnp.dot(x, wr_ref[t], preferred_element_type=jnp.float32))
        if normalize:
            nrm = jnp.sqrt(jnp.sum(o * o, axis=-1, keepdims=True))
            o = o / jnp.maximum(nrm, 1e-12)
        acc = acc + o
    h = acc / 3.0
    if relu_ln:
        h = jax.nn.relu(h)
        mu = jnp.mean(h, axis=-1, keepdims=True)
        var = jnp.mean((h - mu) ** 2, axis=-1, keepdims=True)
        h = (h - mu) / jnp.sqrt(var + 1e-5) * g_ref[0] + b_ref[0]
    h_ref[...] = h
    hs_ref[0] = h[:, :128]
    hs_ref[1] = h[:, 128:]


def _combine(sums, n_chunk, chunked, cnt, x, Wl, bl, Wr, ln_g, ln_b,
             normalize, relu_ln):
    d_in = x.shape[1]
    body = functools.partial(_combine_body, n_chunk, chunked, normalize,
                             relu_ln)
    if chunked:
        sum_specs = [pl.BlockSpec((3, NC, BN, 128), lambda i: (0, 0, i, 0))]
    else:
        sum_specs = [
            pl.BlockSpec((1, 3, NC, BN, 128), lambda i, c=c: (c, 0, 0, i, 0))
            for c in range(n_chunk)]
    return pl.pallas_call(
        body,
        grid=(NB,),
        in_specs=sum_specs + [
            pl.BlockSpec((3, NC, BN, 128), lambda i: (0, 0, i, 0)),
            pl.BlockSpec((BN, d_in), lambda i: (i, 0)),
            pl.BlockSpec((3, d_in, D_H), lambda i: (0, 0, 0)),
            pl.BlockSpec((3, D_H), lambda i: (0, 0)),
            pl.BlockSpec((3, d_in, D_H), lambda i: (0, 0, 0)),
            pl.BlockSpec((1, D_H), lambda i: (0, 0)),
            pl.BlockSpec((1, D_H), lambda i: (0, 0)),
        ],
        out_specs=[
            pl.BlockSpec((BN, D_H), lambda i: (i, 0)),
            pl.BlockSpec((2, BN, 128), lambda i: (0, i, 0)),
        ],
        out_shape=[
            jax.ShapeDtypeStruct((N, D_H), jnp.float32),
            jax.ShapeDtypeStruct((2, N, 128), jnp.float32),
        ],
    )(*([sums] * n_chunk), cnt, x, Wl, bl, Wr, ln_g, ln_b)


def _final_body(s_ref, cnt_ref, x_ref, wl_ref, bl_ref, wr_ref,
                cW1_ref, cb1_ref, cW2_ref, cb2_ref, g_ref, b_ref,
                cW3_ref, cb3_ref, o_ref):
    x = x_ref[...]
    acc = jnp.zeros((BN, D_H), jnp.float32)
    for t in range(3):
        cnt = cnt_ref[t, 0, :, 0] + cnt_ref[t, 1, :, 0]
        cnt = jnp.maximum(cnt, 1.0)
        parts = []
        for c in range(2):
            parts.append(s_ref[t, c] / cnt[:, None])
        aggr = jnp.concatenate(parts, axis=1)
        acc = acc + (
            jnp.dot(aggr, wl_ref[t], preferred_element_type=jnp.float32)
            + bl_ref[t]
            + jnp.dot(x, wr_ref[t], preferred_element_type=jnp.float32))
    h = acc / 3.0
    z = jax.nn.relu(
        jnp.dot(h, cW1_ref[...], preferred_element_type=jnp.float32)
        + cb1_ref[...])
    z = jax.nn.relu(
        jnp.dot(z, cW2_ref[...], preferred_element_type=jnp.float32)
        + cb2_ref[...])
    mu = jnp.mean(z, axis=-1, keepdims=True)
    var = jnp.mean((z - mu) ** 2, axis=-1, keepdims=True)
    z = (z - mu) / jnp.sqrt(var + 1e-5) * g_ref[...] + b_ref[...]
    o_ref[...] = (jnp.dot(z, cW3_ref[...], preferred_element_type=jnp.float32)
                  + cb3_ref[...])


def _final(sums, cnt, h, Wl, bl, Wr, cW1, cb1, cW2, cb2, cg, cb, cW3,
           cb3):
    dh2 = D_H // 2
    return pl.pallas_call(
        _final_body,
        grid=(NB,),
        in_specs=[
            pl.BlockSpec((3, NC, BN, 128), lambda i: (0, 0, i, 0)),
            pl.BlockSpec((3, NC, BN, 128), lambda i: (0, 0, i, 0)),
            pl.BlockSpec((BN, D_H), lambda i: (i, 0)),
            pl.BlockSpec((3, D_H, D_H), lambda i: (0, 0, 0)),
            pl.BlockSpec((3, D_H), lambda i: (0, 0)),
            pl.BlockSpec((3, D_H, D_H), lambda i: (0, 0, 0)),
            pl.BlockSpec((D_H, dh2), lambda i: (0, 0)),
            pl.BlockSpec((dh2,), lambda i: (0,)),
            pl.BlockSpec((dh2, dh2), lambda i: (0, 0)),
            pl.BlockSpec((dh2,), lambda i: (0,)),
            pl.BlockSpec((dh2,), lambda i: (0,)),
            pl.BlockSpec((dh2,), lambda i: (0,)),
            pl.BlockSpec((dh2, N_CLS), lambda i: (0, 0)),
            pl.BlockSpec((N_CLS,), lambda i: (0,)),
        ],
        out_specs=pl.BlockSpec((BN, N_CLS), lambda i: (i, 0)),
        out_shape=jax.ShapeDtypeStruct((N, N_CLS), jnp.float32),
    )(sums, cnt, h, Wl, bl, Wr, cW1, cb1, cW2, cb2, cg, cb, cW3, cb3)


def kernel(x, ei, Wp0, bp0, Wl0, bl0, Wr0, Wl1, bl1, Wr1, Wl2, bl2, Wr2,
           ln_g, ln_b, cW1, cb1, cW2, cb2, cln_g, cln_b, cW3, cb3):
    src = ei[:, 0, :]
    dst = ei[:, 1, :]
    src_pad = jnp.pad(src, ((0, 0), (0, E_PAD - E)))
    dst_pad = jnp.pad(dst, ((0, 0), (0, E_PAD - E)), constant_values=TRASH)
    srcs0 = (src_pad + (jnp.arange(3, dtype=jnp.int32) * N)[:, None])
    srcs0 = srcs0.reshape(3, NC, NS, NSTEP, 1, G)
    srcsH = jnp.stack([src_pad, src_pad + N])
    srcsH = srcsH.reshape(NC, 3, NS, NSTEP2, 1, G)
    dst_blk = dst_pad.reshape(3, NC, NS, NSTEP, 1, G)
    dst_ch = dst_pad.reshape(3, NS, NSTEP2, 1, G)

    zeros = jnp.zeros((N_PAD, 128), jnp.float32)
    ones = jnp.ones((G, 128), jnp.float32)
    cnt = _seg_counts(zeros, ones, dst_blk)

    xs = _project0(x, Wp0, bp0)
    sums0 = _seg_sums(xs.reshape(3 * N, D_IN), zeros, [srcs0], dst_blk)
    h, hs = _combine(sums0, 1, False, cnt, x, Wl0, bl0, Wr0, ln_g[0:1],
                     ln_b[0:1], normalize=True, relu_ln=True)

    hsf = hs.reshape(2 * N, 128)
    sums1 = _seg_sums_ch(hsf, zeros, srcsH, dst_ch)
    h2, h2s = _combine(sums1, 1, True, cnt, h, Wl1, bl1, Wr1, ln_g[1:2],
                       ln_b[1:2], normalize=False, relu_ln=True)

    h2sf = h2s.reshape(2 * N, 128)
    sums2 = _seg_sums_ch(h2sf, zeros, srcsH, dst_ch)
    return _final(sums2, cnt, h2, Wl2, bl2, Wr2, cW1, cb1, cW2,
                  cb2, cln_g, cln_b, cW3, cb3)
